# Initial kernel scaffold; baseline (speedup 1.0000x reference)
import jax
import jax.numpy as jnp
from jax.experimental import pallas as pl


def kernel(x, edge_index, batch_ids, W1, b1, W2, b2, W3, b3, Wlin, blin, Wout, bout):
    raise NotImplementedError("write your pallas kernel here")


# speedup vs baseline: 7.0308x; 7.0308x over previous
import functools

import jax
import jax.numpy as jnp
from jax import lax
from jax.experimental import pallas as pl
from jax.experimental.pallas import tpu as pltpu
from jax.experimental.pallas import tpu_sc as plsc

N = 10000
NP = 10240
E = 320000
NC, NS = 2, 16
NW = NC * NS
EPW = 10240
EP = NW * EPW
KB = 128
NBLK = EPW // KB
RPT = NP // NS
G = 64
C = 10
BLK = 1024
GRID = NP // BLK
NEG = 0.01

_mesh = plsc.VectorSubcoreMesh(core_axis_name="c", subcore_axis_name="s")


def _fill(ref, nrows, ncols, val):
    v = jnp.full((16,), val, jnp.float32)

    def body(i, _):
        for j in range(ncols // 16):
            ref[i, pl.ds(j * 16, 16)] = v
        return 0

    lax.fori_loop(0, nrows, body, 0)


@functools.partial(
    pl.kernel,
    out_type=jax.ShapeDtypeStruct((2 * NP, 16), jnp.float32),
    mesh=_mesh,
    scratch_types=[
        pltpu.VMEM((KB,), jnp.int32),
        pltpu.VMEM((KB, 16), jnp.float32),
        pltpu.VMEM_SHARED((NP, 16), jnp.float32),
    ],
)
def _sc_degree(dst_hbm, out_hbm, didx, buf, acc):
    c = lax.axis_index("c")
    s = lax.axis_index("s")
    wid = c * NS + s

    _fill(buf, KB, 16, 0.0)
    for t in range(RPT // KB):
        pltpu.sync_copy(buf, acc.at[pl.ds(s * RPT + t * KB, KB)])
    _fill(buf, KB, 16, 1.0)
    plsc.subcore_barrier()

    ebase = wid * EPW

    def blk(b, _):
        pltpu.sync_copy(dst_hbm.at[pl.ds(ebase + b * KB, KB)], didx)
        pltpu.sync_copy(buf, acc.at[didx], add=True)
        return 0

    lax.fori_loop(0, NBLK, blk, 0)
    plsc.subcore_barrier()
    pltpu.sync_copy(acc.at[pl.ds(s * RPT, RPT)],
                    out_hbm.at[pl.ds(c * NP + s * RPT, RPT)])


@functools.partial(
    pl.kernel,
    out_type=jax.ShapeDtypeStruct((2 * NP, 128), jnp.float32),
    mesh=_mesh,
    scratch_types=[
        pltpu.VMEM((KB,), jnp.int32),
        pltpu.VMEM((KB,), jnp.int32),
        pltpu.VMEM((KB, 128), jnp.float32),
        pltpu.VMEM_SHARED((NP, 128), jnp.float32),
        pltpu.SemaphoreType.DMA,
    ],
)
def _sc_edge_pass(m_hbm, src_hbm, dst_hbm, out_hbm, sidx, didx, rows, acc, sem):
    c = lax.axis_index("c")
    s = lax.axis_index("s")
    wid = c * NS + s

    _fill(rows, KB, 128, 0.0)
    for t in range(RPT // KB):
        pltpu.sync_copy(rows, acc.at[pl.ds(s * RPT + t * KB, KB)])
    plsc.subcore_barrier()

    ebase = wid * EPW

    def blk(b, _):
        base = ebase + b * KB
        pltpu.sync_copy(src_hbm.at[pl.ds(base, KB)], sidx)
        pltpu.sync_copy(dst_hbm.at[pl.ds(base, KB)], didx)
        pltpu.async_copy(m_hbm.at[sidx], rows, sem).wait()
        pltpu.sync_copy(rows, acc.at[didx], add=True)
        return 0

    lax.fori_loop(0, NBLK, blk, 0)
    plsc.subcore_barrier()
    pltpu.sync_copy(acc.at[pl.ds(s * RPT, RPT)],
                    out_hbm.at[pl.ds(c * NP + s * RPT, RPT)])


def _t1_body(x_ref, w_ref, d0_ref, d1_ref, m_ref, dv_ref):
    i = pl.program_id(0)
    deg = d0_ref[:, 0:1] + d1_ref[:, 0:1] + 1.0
    rows = i * BLK + lax.broadcasted_iota(jnp.int32, (BLK, 1), 0)
    dinv = jnp.where(rows < N, lax.rsqrt(deg), 0.0)
    dv = jnp.broadcast_to(dinv, (BLK, 128))
    dv_ref[...] = dv
    m_ref[...] = jnp.dot(x_ref[...], w_ref[...],
                         preferred_element_type=jnp.float32) * dv


def _t1(x, w1, degp0, degp1):
    return pl.pallas_call(
        _t1_body,
        grid=(GRID,),
        in_specs=[
            pl.BlockSpec((BLK, 128), lambda i: (i, 0)),
            pl.BlockSpec((128, 128), lambda i: (0, 0)),
            pl.BlockSpec((BLK, 16), lambda i: (i, 0)),
            pl.BlockSpec((BLK, 16), lambda i: (i, 0)),
        ],
        out_specs=[
            pl.BlockSpec((BLK, 128), lambda i: (i, 0)),
            pl.BlockSpec((BLK, 128), lambda i: (i, 0)),
        ],
        out_shape=[
            jax.ShapeDtypeStruct((NP, 128), jnp.float32),
            jax.ShapeDtypeStruct((NP, 128), jnp.float32),
        ],
    )(x, w1, degp0, degp1)


def _t2_body(p0_ref, p1_ref, m_ref, dv_ref, b_ref, w_ref, out_ref):
    h = (p0_ref[...] + p1_ref[...] + m_ref[...]) * dv_ref[...] + b_ref[...]
    h = jnp.where(h >= 0, h, NEG * h)
    out_ref[...] = jnp.dot(h, w_ref[...],
                           preferred_element_type=jnp.float32) * dv_ref[...]


def _t2(p0, p1, m, dv, b, w):
    return pl.pallas_call(
        _t2_body,
        grid=(GRID,),
        in_specs=[
            pl.BlockSpec((BLK, 128), lambda i: (i, 0)),
            pl.BlockSpec((BLK, 128), lambda i: (i, 0)),
            pl.BlockSpec((BLK, 128), lambda i: (i, 0)),
            pl.BlockSpec((BLK, 128), lambda i: (i, 0)),
            pl.BlockSpec((1, 128), lambda i: (0, 0)),
            pl.BlockSpec((128, 128), lambda i: (0, 0)),
        ],
        out_specs=pl.BlockSpec((BLK, 128), lambda i: (i, 0)),
        out_shape=jax.ShapeDtypeStruct((NP, 128), jnp.float32),
    )(p0, p1, m, dv, b, w)


def _t3_body(p0_ref, p1_ref, m_ref, dv_ref, b_ref, bid_ref, wl_ref, bl_ref,
             wo_ref, bo_ref, lg_ref, pr_ref, em_ref, sp_acc, ct_acc):
    i = pl.program_id(0)

    @pl.when(i == 0)
    def _():
        sp_acc[...] = jnp.zeros((G, 128), jnp.float32)
        ct_acc[...] = jnp.zeros((G, 128), jnp.float32)

    h3 = (p0_ref[...] + p1_ref[...] + m_ref[...]) * dv_ref[...] + b_ref[...]
    bid = bid_ref[0, 0, :]
    oh = (lax.broadcasted_iota(jnp.int32, (G, BLK), 0)
          == bid[None, :]).astype(jnp.float32)
    sp_acc[...] += jnp.dot(oh, h3, preferred_element_type=jnp.float32)
    ct_acc[...] += jnp.broadcast_to(
        jnp.sum(oh, axis=1, keepdims=True), (G, 128))

    @pl.when(i == GRID - 1)
    def _():
        sp = sp_acc[...]
        cnt = ct_acc[...][:, 0:1]
        mp = sp / jnp.maximum(cnt, 1.0)
        em = jnp.concatenate([sp, mp], axis=1)
        em_ref[...] = em
        h2 = jnp.dot(em, wl_ref[...],
                     preferred_element_type=jnp.float32) + bl_ref[...]
        h2 = jnp.maximum(h2, 0.0)
        lg = jnp.dot(h2, wo_ref[...],
                     preferred_element_type=jnp.float32) + bo_ref[...]
        lg_ref[...] = lg
        col = lax.broadcasted_iota(jnp.int32, (G, 128), 1)
        lgm = jnp.where(col < C, lg, -1e30)
        mx = jnp.max(lgm, axis=1, keepdims=True)
        ex = jnp.exp(lgm - mx)
        pr_ref[...] = ex / jnp.sum(ex, axis=1, keepdims=True)


def _t3(p0, p1, m, dv, b3, bid3, wl, bl, wo, bo):
    return pl.pallas_call(
        _t3_body,
        grid=(GRID,),
        in_specs=[
            pl.BlockSpec((BLK, 128), lambda i: (i, 0)),
            pl.BlockSpec((BLK, 128), lambda i: (i, 0)),
            pl.BlockSpec((BLK, 128), lambda i: (i, 0)),
            pl.BlockSpec((BLK, 128), lambda i: (i, 0)),
            pl.BlockSpec((1, 128), lambda i: (0, 0)),
            pl.BlockSpec((1, 1, BLK), lambda i: (i, 0, 0)),
            pl.BlockSpec((256, 128), lambda i: (0, 0)),
            pl.BlockSpec((1, 128), lambda i: (0, 0)),
            pl.BlockSpec((128, 128), lambda i: (0, 0)),
            pl.BlockSpec((1, 128), lambda i: (0, 0)),
        ],
        out_specs=[
            pl.BlockSpec((G, 128), lambda i: (0, 0)),
            pl.BlockSpec((G, 128), lambda i: (0, 0)),
            pl.BlockSpec((G, 256), lambda i: (0, 0)),
        ],
        out_shape=[
            jax.ShapeDtypeStruct((G, 128), jnp.float32),
            jax.ShapeDtypeStruct((G, 128), jnp.float32),
            jax.ShapeDtypeStruct((G, 256), jnp.float32),
        ],
        scratch_shapes=[
            pltpu.VMEM((G, 128), jnp.float32),
            pltpu.VMEM((G, 128), jnp.float32),
        ],
    )(p0, p1, m, dv, b3, bid3, wl, bl, wo, bo)


def kernel(x, edge_index, batch_ids, W1, b1, W2, b2, W3, b3,
           Wlin, blin, Wout, bout):
    f32 = jnp.float32
    src = edge_index[0].astype(jnp.int32)
    dst = edge_index[1].astype(jnp.int32)
    pad = jnp.full((EP - E,), N, jnp.int32)
    src_p = jnp.concatenate([src, pad])
    dst_p = jnp.concatenate([dst, pad])

    x_p = jnp.zeros((NP, 128), f32).at[:N].set(x)
    bid_p = jnp.full((NP,), G, jnp.int32).at[:N].set(batch_ids.astype(jnp.int32))
    bid3 = bid_p.reshape(GRID, 1, BLK)

    b1r = b1.reshape(1, 128)
    b2r = b2.reshape(1, 128)
    b3r = b3.reshape(1, 128)
    blr = blin.reshape(1, 128)
    wo_p = jnp.zeros((128, 128), f32).at[:, :C].set(Wout)
    bo_p = jnp.zeros((1, 128), f32).at[0, :C].set(bout)

    degp = _sc_degree(dst_p)
    m1, dv = _t1(x_p, W1, degp[:NP], degp[NP:])
    p1_ = _sc_edge_pass(m1, src_p, dst_p)
    m2 = _t2(p1_[:NP], p1_[NP:], m1, dv, b1r, W2)
    p2_ = _sc_edge_pass(m2, src_p, dst_p)
    m3 = _t2(p2_[:NP], p2_[NP:], m2, dv, b2r, W3)
    p3_ = _sc_edge_pass(m3, src_p, dst_p)
    lg, pr, em = _t3(p3_[:NP], p3_[NP:], m3, dv, b3r, bid3, Wlin, blr,
                     wo_p, bo_p)
    return lg[:, :C], pr[:, :C], em


# speedup vs baseline: 7.2090x; 1.0253x over previous
import functools

import jax
import jax.numpy as jnp
from jax import lax
from jax.experimental import pallas as pl
from jax.experimental.pallas import tpu as pltpu
from jax.experimental.pallas import tpu_sc as plsc

N = 10000
NP = 10240
E = 320000
NC, NS = 2, 16
NW = NC * NS
EPW = 10240
EP = NW * EPW
KB = 128
NBLK = EPW // KB
RPT = NP // NS
G = 64
C = 10
BLK = 1024
GRID = NP // BLK
NEG = 0.01

_mesh = plsc.VectorSubcoreMesh(core_axis_name="c", subcore_axis_name="s")


def _fill(ref, nrows, ncols, val):
    v = jnp.full((16,), val, jnp.float32)

    def body(i, _):
        for j in range(ncols // 16):
            ref[i, pl.ds(j * 16, 16)] = v
        return 0

    lax.fori_loop(0, nrows, body, 0)


SHIFT = 14
MASK = (1 << SHIFT) - 1


def _unpack(packed, b, sidx, didx):
    for j in range(KB // 16):
        v = packed[b, pl.ds(j * 16, 16)]
        if sidx is not None:
            sidx[pl.ds(j * 16, 16)] = lax.shift_right_logical(v, SHIFT)
        didx[pl.ds(j * 16, 16)] = lax.bitwise_and(v, MASK)


@functools.partial(
    pl.kernel,
    out_type=jax.ShapeDtypeStruct((2 * NP, 16), jnp.float32),
    mesh=_mesh,
    scratch_types=[
        pltpu.VMEM((NBLK, KB), jnp.int32),
        pltpu.VMEM((KB,), jnp.int32),
        pltpu.VMEM((KB,), jnp.int32),
        pltpu.VMEM((KB, 16), jnp.float32),
        pltpu.VMEM_SHARED((NP, 16), jnp.float32),
        pltpu.SemaphoreType.DMA,
        pltpu.SemaphoreType.DMA,
    ],
)
def _sc_degree(edges_hbm, out_hbm, packed, didx0, didx1, buf, acc, s0, s1):
    c = lax.axis_index("c")
    s = lax.axis_index("s")
    wid = c * NS + s
    ssem = (s0, s1)
    didx = (didx0, didx1)

    pltpu.sync_copy(edges_hbm.at[wid], packed)
    _fill(buf, KB, 16, 0.0)
    for t in range(RPT // KB):
        pltpu.sync_copy(buf, acc.at[pl.ds(s * RPT + t * KB, KB)])
    _fill(buf, KB, 16, 1.0)
    plsc.subcore_barrier()

    def scat(j):
        pltpu.async_copy(buf, acc.at[didx[j]], ssem[j], add=True)

    def swait(j):
        pltpu.make_async_copy(buf, acc.at[didx[j]], ssem[j]).wait()

    def blk(b, _):
        _unpack(packed, b, None, didx[0])
        scat(0)
        swait(0)
        return 0

    lax.fori_loop(0, NBLK, blk, 0)
    plsc.subcore_barrier()
    pltpu.sync_copy(acc.at[pl.ds(s * RPT, RPT)],
                    out_hbm.at[pl.ds(c * NP + s * RPT, RPT)])


@functools.partial(
    pl.kernel,
    out_type=jax.ShapeDtypeStruct((2 * NP, 128), jnp.float32),
    mesh=_mesh,
    scratch_types=[
        pltpu.VMEM((NBLK, KB), jnp.int32),
        pltpu.VMEM((KB,), jnp.int32),
        pltpu.VMEM((KB,), jnp.int32),
        pltpu.VMEM((KB,), jnp.int32),
        pltpu.VMEM((KB,), jnp.int32),
        pltpu.VMEM((2, KB, 128), jnp.float32),
        pltpu.VMEM_SHARED((NP, 128), jnp.float32),
        pltpu.SemaphoreType.DMA,
        pltpu.SemaphoreType.DMA,
        pltpu.SemaphoreType.DMA,
        pltpu.SemaphoreType.DMA,
    ],
)
def _sc_edge_pass(m_hbm, edges_hbm, out_hbm, packed, sidx0, sidx1,
                  didx0, didx1, rows, acc, g0, g1, s0, s1):
    c = lax.axis_index("c")
    s = lax.axis_index("s")
    wid = c * NS + s
    gsem = (g0, g1)
    ssem = (s0, s1)
    sidx = (sidx0, sidx1)
    didx = (didx0, didx1)

    pltpu.sync_copy(edges_hbm.at[wid], packed)
    _fill(rows.at[0], KB, 128, 0.0)
    for t in range(RPT // KB):
        pltpu.sync_copy(rows.at[0], acc.at[pl.ds(s * RPT + t * KB, KB)])
    plsc.subcore_barrier()

    def gath(j):
        pltpu.async_copy(m_hbm.at[sidx[j]], rows.at[j], gsem[j])

    def gwait(j):
        pltpu.make_async_copy(m_hbm.at[sidx[j]], rows.at[j], gsem[j]).wait()

    def scat(j):
        pltpu.async_copy(rows.at[j], acc.at[didx[j]], ssem[j], add=True)

    def swait(j):
        pltpu.make_async_copy(rows.at[j], acc.at[didx[j]], ssem[j]).wait()

    def scat_sync(j):
        scat(j)
        swait(j)

    _unpack(packed, 0, sidx[0], didx[0])
    gath(0)

    def blk(i, _):
        for j in range(2):
            b = 2 * i + j
            _unpack(packed, b + 1, sidx[1 - j], didx[1 - j])
            gwait(j)
            gath(1 - j)
            scat_sync(j)
        return 0

    lax.fori_loop(0, NBLK // 2 - 1, blk, 0)

    _unpack(packed, NBLK - 1, sidx[1], didx[1])
    gwait(0)
    gath(1)
    scat_sync(0)
    gwait(1)
    scat_sync(1)

    plsc.subcore_barrier()
    pltpu.sync_copy(acc.at[pl.ds(s * RPT, RPT)],
                    out_hbm.at[pl.ds(c * NP + s * RPT, RPT)])


def _t1_body(x_ref, w_ref, d0_ref, d1_ref, m_ref, dv_ref):
    i = pl.program_id(0)
    deg = d0_ref[:, 0:1] + d1_ref[:, 0:1] + 1.0
    rows = i * BLK + lax.broadcasted_iota(jnp.int32, (BLK, 1), 0)
    dinv = jnp.where(rows < N, lax.rsqrt(deg), 0.0)
    dv = jnp.broadcast_to(dinv, (BLK, 128))
    dv_ref[...] = dv
    m_ref[...] = jnp.dot(x_ref[...], w_ref[...],
                         preferred_element_type=jnp.float32) * dv


def _t1(x, w1, degp0, degp1):
    return pl.pallas_call(
        _t1_body,
        grid=(GRID,),
        in_specs=[
            pl.BlockSpec((BLK, 128), lambda i: (i, 0)),
            pl.BlockSpec((128, 128), lambda i: (0, 0)),
            pl.BlockSpec((BLK, 16), lambda i: (i, 0)),
            pl.BlockSpec((BLK, 16), lambda i: (i, 0)),
        ],
        out_specs=[
            pl.BlockSpec((BLK, 128), lambda i: (i, 0)),
            pl.BlockSpec((BLK, 128), lambda i: (i, 0)),
        ],
        out_shape=[
            jax.ShapeDtypeStruct((NP, 128), jnp.float32),
            jax.ShapeDtypeStruct((NP, 128), jnp.float32),
        ],
    )(x, w1, degp0, degp1)


def _t2_body(p0_ref, p1_ref, m_ref, dv_ref, b_ref, w_ref, out_ref):
    h = (p0_ref[...] + p1_ref[...] + m_ref[...]) * dv_ref[...] + b_ref[...]
    h = jnp.where(h >= 0, h, NEG * h)
    out_ref[...] = jnp.dot(h, w_ref[...],
                           preferred_element_type=jnp.float32) * dv_ref[...]


def _t2(p0, p1, m, dv, b, w):
    return pl.pallas_call(
        _t2_body,
        grid=(GRID,),
        in_specs=[
            pl.BlockSpec((BLK, 128), lambda i: (i, 0)),
            pl.BlockSpec((BLK, 128), lambda i: (i, 0)),
            pl.BlockSpec((BLK, 128), lambda i: (i, 0)),
            pl.BlockSpec((BLK, 128), lambda i: (i, 0)),
            pl.BlockSpec((1, 128), lambda i: (0, 0)),
            pl.BlockSpec((128, 128), lambda i: (0, 0)),
        ],
        out_specs=pl.BlockSpec((BLK, 128), lambda i: (i, 0)),
        out_shape=jax.ShapeDtypeStruct((NP, 128), jnp.float32),
    )(p0, p1, m, dv, b, w)


def _t3_body(p0_ref, p1_ref, m_ref, dv_ref, b_ref, bid_ref, wl_ref, bl_ref,
             wo_ref, bo_ref, lg_ref, pr_ref, em_ref, sp_acc, ct_acc):
    i = pl.program_id(0)

    @pl.when(i == 0)
    def _():
        sp_acc[...] = jnp.zeros((G, 128), jnp.float32)
        ct_acc[...] = jnp.zeros((G, 128), jnp.float32)

    h3 = (p0_ref[...] + p1_ref[...] + m_ref[...]) * dv_ref[...] + b_ref[...]
    bid = bid_ref[0, 0, :]
    oh = (lax.broadcasted_iota(jnp.int32, (G, BLK), 0)
          == bid[None, :]).astype(jnp.float32)
    sp_acc[...] += jnp.dot(oh, h3, preferred_element_type=jnp.float32)
    ct_acc[...] += jnp.broadcast_to(
        jnp.sum(oh, axis=1, keepdims=True), (G, 128))

    @pl.when(i == GRID - 1)
    def _():
        sp = sp_acc[...]
        cnt = ct_acc[...][:, 0:1]
        mp = sp / jnp.maximum(cnt, 1.0)
        em = jnp.concatenate([sp, mp], axis=1)
        em_ref[...] = em
        h2 = jnp.dot(em, wl_ref[...],
                     preferred_element_type=jnp.float32) + bl_ref[...]
        h2 = jnp.maximum(h2, 0.0)
        lg = jnp.dot(h2, wo_ref[...],
                     preferred_element_type=jnp.float32) + bo_ref[...]
        lg_ref[...] = lg
        col = lax.broadcasted_iota(jnp.int32, (G, 128), 1)
        lgm = jnp.where(col < C, lg, -1e30)
        mx = jnp.max(lgm, axis=1, keepdims=True)
        ex = jnp.exp(lgm - mx)
        pr_ref[...] = ex / jnp.sum(ex, axis=1, keepdims=True)


def _t3(p0, p1, m, dv, b3, bid3, wl, bl, wo, bo):
    return pl.pallas_call(
        _t3_body,
        grid=(GRID,),
        in_specs=[
            pl.BlockSpec((BLK, 128), lambda i: (i, 0)),
            pl.BlockSpec((BLK, 128), lambda i: (i, 0)),
            pl.BlockSpec((BLK, 128), lambda i: (i, 0)),
            pl.BlockSpec((BLK, 128), lambda i: (i, 0)),
            pl.BlockSpec((1, 128), lambda i: (0, 0)),
            pl.BlockSpec((1, 1, BLK), lambda i: (i, 0, 0)),
            pl.BlockSpec((256, 128), lambda i: (0, 0)),
            pl.BlockSpec((1, 128), lambda i: (0, 0)),
            pl.BlockSpec((128, 128), lambda i: (0, 0)),
            pl.BlockSpec((1, 128), lambda i: (0, 0)),
        ],
        out_specs=[
            pl.BlockSpec((G, 128), lambda i: (0, 0)),
            pl.BlockSpec((G, 128), lambda i: (0, 0)),
            pl.BlockSpec((G, 256), lambda i: (0, 0)),
        ],
        out_shape=[
            jax.ShapeDtypeStruct((G, 128), jnp.float32),
            jax.ShapeDtypeStruct((G, 128), jnp.float32),
            jax.ShapeDtypeStruct((G, 256), jnp.float32),
        ],
        scratch_shapes=[
            pltpu.VMEM((G, 128), jnp.float32),
            pltpu.VMEM((G, 128), jnp.float32),
        ],
    )(p0, p1, m, dv, b3, bid3, wl, bl, wo, bo)


def kernel(x, edge_index, batch_ids, W1, b1, W2, b2, W3, b3,
           Wlin, blin, Wout, bout):
    f32 = jnp.float32
    src = edge_index[0].astype(jnp.int32)
    dst = edge_index[1].astype(jnp.int32)
    packed = src * (1 << SHIFT) + dst
    padv = jnp.full((EP - E,), N * (1 << SHIFT) + N, jnp.int32)
    edges = jnp.concatenate([packed, padv]).reshape(NW, NBLK, KB)

    x_p = jnp.zeros((NP, 128), f32).at[:N].set(x)
    bid_p = jnp.full((NP,), G, jnp.int32).at[:N].set(batch_ids.astype(jnp.int32))
    bid3 = bid_p.reshape(GRID, 1, BLK)

    b1r = b1.reshape(1, 128)
    b2r = b2.reshape(1, 128)
    b3r = b3.reshape(1, 128)
    blr = blin.reshape(1, 128)
    wo_p = jnp.zeros((128, 128), f32).at[:, :C].set(Wout)
    bo_p = jnp.zeros((1, 128), f32).at[0, :C].set(bout)

    degp = _sc_degree(edges)
    m1, dv = _t1(x_p, W1, degp[:NP], degp[NP:])
    p1_ = _sc_edge_pass(m1, edges)
    m2 = _t2(p1_[:NP], p1_[NP:], m1, dv, b1r, W2)
    p2_ = _sc_edge_pass(m2, edges)
    m3 = _t2(p2_[:NP], p2_[NP:], m2, dv, b2r, W3)
    p3_ = _sc_edge_pass(m3, edges)
    lg, pr, em = _t3(p3_[:NP], p3_[NP:], m3, dv, b3r, bid3, Wlin, blr,
                     wo_p, bo_p)
    return lg[:, :C], pr[:, :C], em


# speedup vs baseline: 24.4837x; 3.3963x over previous
import functools

import jax
import jax.numpy as jnp
from jax import lax
from jax.experimental import pallas as pl
from jax.experimental.pallas import tpu as pltpu
from jax.experimental.pallas import tpu_sc as plsc

N = 10000
NP = 10240
E = 320000
NC, NS = 2, 16
NW = NC * NS
EPW = 10240
EP = NW * EPW
KB = 128
NBLK = EPW // KB
RPT = NP // NS
G = 64
C = 10
BLK = 1024
GRID = NP // BLK
NEG = 0.01
SHIFT = 14
MASK = (1 << SHIFT) - 1

_mesh = plsc.VectorSubcoreMesh(core_axis_name="c", subcore_axis_name="s")


def _fill(ref, nrows, ncols, val):
    v = jnp.full((16,), val, jnp.float32)

    def body(i, _):
        for j in range(ncols // 16):
            ref[i, pl.ds(j * 16, 16)] = v
        return 0

    lax.fori_loop(0, nrows, body, 0)


def _unpack(packed, b, sidx, didx):
    for j in range(KB // 16):
        v = packed[b, pl.ds(j * 16, 16)]
        if sidx is not None:
            sidx[pl.ds(j * 16, 16)] = lax.shift_right_logical(v, SHIFT)
        didx[pl.ds(j * 16, 16)] = lax.bitwise_and(v, MASK)


@functools.partial(
    pl.kernel,
    out_type=jax.ShapeDtypeStruct((2 * NP, 16), jnp.float32),
    mesh=_mesh,
    scratch_types=[
        pltpu.VMEM((NBLK, KB), jnp.int32),
        pltpu.VMEM((KB,), jnp.int32),
        pltpu.VMEM((KB,), jnp.int32),
        pltpu.VMEM((KB, 16), jnp.float32),
        pltpu.VMEM_SHARED((NP, 16), jnp.float32),
        pltpu.SemaphoreType.DMA,
        pltpu.SemaphoreType.DMA,
    ],
)
def _sc_degree(edges_hbm, out_hbm, packed, didx0, didx1, buf, acc, s0, s1):
    c = lax.axis_index("c")
    s = lax.axis_index("s")
    wid = c * NS + s
    ssem = (s0, s1)
    didx = (didx0, didx1)

    pltpu.sync_copy(edges_hbm.at[wid], packed)
    _fill(buf, KB, 16, 0.0)
    for t in range(RPT // KB):
        pltpu.sync_copy(buf, acc.at[pl.ds(s * RPT + t * KB, KB)])
    _fill(buf, KB, 16, 1.0)
    plsc.subcore_barrier()

    def scat(j):
        pltpu.async_copy(buf, acc.at[didx[j]], ssem[j], add=True)

    def swait(j):
        pltpu.make_async_copy(buf, acc.at[didx[j]], ssem[j]).wait()

    _unpack(packed, 0, None, didx[0])
    scat(0)
    _unpack(packed, 1, None, didx[1])
    scat(1)

    def blk(i, _):
        for j in range(2):
            b = 2 * i + j
            swait(j)
            _unpack(packed, b, None, didx[j])
            scat(j)
        return 0

    lax.fori_loop(1, NBLK // 2, blk, 0)
    swait(0)
    swait(1)
    plsc.subcore_barrier()
    pltpu.sync_copy(acc.at[pl.ds(s * RPT, RPT)],
                    out_hbm.at[pl.ds(c * NP + s * RPT, RPT)])


@functools.partial(
    pl.kernel,
    out_type=jax.ShapeDtypeStruct((2 * NP, 128), jnp.float32),
    mesh=_mesh,
    scratch_types=[
        pltpu.VMEM((NBLK, KB), jnp.int32),
        pltpu.VMEM((KB,), jnp.int32),
        pltpu.VMEM((KB,), jnp.int32),
        pltpu.VMEM((KB,), jnp.int32),
        pltpu.VMEM((KB,), jnp.int32),
        pltpu.VMEM((2, KB, 128), jnp.float32),
        pltpu.VMEM_SHARED((NP, 128), jnp.float32),
        pltpu.SemaphoreType.DMA,
        pltpu.SemaphoreType.DMA,
        pltpu.SemaphoreType.DMA,
        pltpu.SemaphoreType.DMA,
    ],
)
def _sc_edge_pass(m_hbm, edges_hbm, out_hbm, packed, sidx0, sidx1,
                  didx0, didx1, rows, acc, g0, g1, s0, s1):
    c = lax.axis_index("c")
    s = lax.axis_index("s")
    wid = c * NS + s
    gsem = (g0, g1)
    ssem = (s0, s1)
    sidx = (sidx0, sidx1)
    didx = (didx0, didx1)

    pltpu.sync_copy(edges_hbm.at[wid], packed)
    _fill(rows.at[0], KB, 128, 0.0)
    for t in range(RPT // KB):
        pltpu.sync_copy(rows.at[0], acc.at[pl.ds(s * RPT + t * KB, KB)])
    plsc.subcore_barrier()

    def gath(j):
        pltpu.async_copy(m_hbm.at[sidx[j]], rows.at[j], gsem[j])

    def gwait(j):
        pltpu.make_async_copy(m_hbm.at[sidx[j]], rows.at[j], gsem[j]).wait()

    def scat_sync(j):
        pltpu.async_copy(rows.at[j], acc.at[didx[j]], ssem[j], add=True)
        pltpu.make_async_copy(rows.at[j], acc.at[didx[j]], ssem[j]).wait()

    _unpack(packed, 0, sidx[0], didx[0])
    gath(0)

    def blk(i, _):
        for j in range(2):
            b = 2 * i + j
            _unpack(packed, b + 1, sidx[1 - j], didx[1 - j])
            gwait(j)
            gath(1 - j)
            scat_sync(j)
        return 0

    lax.fori_loop(0, NBLK // 2 - 1, blk, 0)

    _unpack(packed, NBLK - 1, sidx[1], didx[1])
    gwait(0)
    gath(1)
    scat_sync(0)
    gwait(1)
    scat_sync(1)

    plsc.subcore_barrier()
    pltpu.sync_copy(acc.at[pl.ds(s * RPT, RPT)],
                    out_hbm.at[pl.ds(c * NP + s * RPT, RPT)])


def _t1_body(x_ref, w_ref, d0_ref, d1_ref, m_ref, dv_ref):
    i = pl.program_id(0)
    deg = d0_ref[:, 0:1] + d1_ref[:, 0:1] + 1.0
    rows = i * BLK + lax.broadcasted_iota(jnp.int32, (BLK, 1), 0)
    dinv = jnp.where(rows < N, lax.rsqrt(deg), 0.0)
    dv = jnp.broadcast_to(dinv, (BLK, 128))
    dv_ref[...] = dv
    m_ref[...] = jnp.dot(x_ref[...], w_ref[...],
                         preferred_element_type=jnp.float32) * dv


def _t1(x, w1, degp0, degp1):
    return pl.pallas_call(
        _t1_body,
        grid=(GRID,),
        in_specs=[
            pl.BlockSpec((BLK, 128), lambda i: (i, 0)),
            pl.BlockSpec((128, 128), lambda i: (0, 0)),
            pl.BlockSpec((BLK, 16), lambda i: (i, 0)),
            pl.BlockSpec((BLK, 16), lambda i: (i, 0)),
        ],
        out_specs=[
            pl.BlockSpec((BLK, 128), lambda i: (i, 0)),
            pl.BlockSpec((BLK, 128), lambda i: (i, 0)),
        ],
        out_shape=[
            jax.ShapeDtypeStruct((NP, 128), jnp.float32),
            jax.ShapeDtypeStruct((NP, 128), jnp.float32),
        ],
    )(x, w1, degp0, degp1)


def _t2_body(p0_ref, p1_ref, m_ref, dv_ref, b_ref, w_ref, out_ref):
    h = (p0_ref[...] + p1_ref[...] + m_ref[...]) * dv_ref[...] + b_ref[...]
    h = jnp.where(h >= 0, h, NEG * h)
    out_ref[...] = jnp.dot(h, w_ref[...],
                           preferred_element_type=jnp.float32) * dv_ref[...]


def _t2(p0, p1, m, dv, b, w):
    return pl.pallas_call(
        _t2_body,
        grid=(GRID,),
        in_specs=[
            pl.BlockSpec((BLK, 128), lambda i: (i, 0)),
            pl.BlockSpec((BLK, 128), lambda i: (i, 0)),
            pl.BlockSpec((BLK, 128), lambda i: (i, 0)),
            pl.BlockSpec((BLK, 128), lambda i: (i, 0)),
            pl.BlockSpec((1, 128), lambda i: (0, 0)),
            pl.BlockSpec((128, 128), lambda i: (0, 0)),
        ],
        out_specs=pl.BlockSpec((BLK, 128), lambda i: (i, 0)),
        out_shape=jax.ShapeDtypeStruct((NP, 128), jnp.float32),
    )(p0, p1, m, dv, b, w)


def _t3_body(p0_ref, p1_ref, m_ref, dv_ref, b_ref, bid_ref, wl_ref, bl_ref,
             wo_ref, bo_ref, lg_ref, pr_ref, em_ref, sp_acc, ct_acc):
    i = pl.program_id(0)

    @pl.when(i == 0)
    def _():
        sp_acc[...] = jnp.zeros((G, 128), jnp.float32)
        ct_acc[...] = jnp.zeros((G, 128), jnp.float32)

    h3 = (p0_ref[...] + p1_ref[...] + m_ref[...]) * dv_ref[...] + b_ref[...]
    bid = bid_ref[0, 0, :]
    oh = (lax.broadcasted_iota(jnp.int32, (G, BLK), 0)
          == bid[None, :]).astype(jnp.float32)
    sp_acc[...] += jnp.dot(oh, h3, preferred_element_type=jnp.float32)
    ct_acc[...] += jnp.broadcast_to(
        jnp.sum(oh, axis=1, keepdims=True), (G, 128))

    @pl.when(i == GRID - 1)
    def _():
        sp = sp_acc[...]
        cnt = ct_acc[...][:, 0:1]
        mp = sp / jnp.maximum(cnt, 1.0)
        em = jnp.concatenate([sp, mp], axis=1)
        em_ref[...] = em
        h2 = jnp.dot(em, wl_ref[...],
                     preferred_element_type=jnp.float32) + bl_ref[...]
        h2 = jnp.maximum(h2, 0.0)
        lg = jnp.dot(h2, wo_ref[...],
                     preferred_element_type=jnp.float32) + bo_ref[...]
        lg_ref[...] = lg
        col = lax.broadcasted_iota(jnp.int32, (G, 128), 1)
        lgm = jnp.where(col < C, lg, -1e30)
        mx = jnp.max(lgm, axis=1, keepdims=True)
        ex = jnp.exp(lgm - mx)
        pr_ref[...] = ex / jnp.sum(ex, axis=1, keepdims=True)


def _t3(p0, p1, m, dv, b3, bid3, wl, bl, wo, bo):
    return pl.pallas_call(
        _t3_body,
        grid=(GRID,),
        in_specs=[
            pl.BlockSpec((BLK, 128), lambda i: (i, 0)),
            pl.BlockSpec((BLK, 128), lambda i: (i, 0)),
            pl.BlockSpec((BLK, 128), lambda i: (i, 0)),
            pl.BlockSpec((BLK, 128), lambda i: (i, 0)),
            pl.BlockSpec((1, 128), lambda i: (0, 0)),
            pl.BlockSpec((1, 1, BLK), lambda i: (i, 0, 0)),
            pl.BlockSpec((256, 128), lambda i: (0, 0)),
            pl.BlockSpec((1, 128), lambda i: (0, 0)),
            pl.BlockSpec((128, 128), lambda i: (0, 0)),
            pl.BlockSpec((1, 128), lambda i: (0, 0)),
        ],
        out_specs=[
            pl.BlockSpec((G, 128), lambda i: (0, 0)),
            pl.BlockSpec((G, 128), lambda i: (0, 0)),
            pl.BlockSpec((G, 256), lambda i: (0, 0)),
        ],
        out_shape=[
            jax.ShapeDtypeStruct((G, 128), jnp.float32),
            jax.ShapeDtypeStruct((G, 128), jnp.float32),
            jax.ShapeDtypeStruct((G, 256), jnp.float32),
        ],
        scratch_shapes=[
            pltpu.VMEM((G, 128), jnp.float32),
            pltpu.VMEM((G, 128), jnp.float32),
        ],
    )(p0, p1, m, dv, b3, bid3, wl, bl, wo, bo)


def kernel(x, edge_index, batch_ids, W1, b1, W2, b2, W3, b3,
           Wlin, blin, Wout, bout):
    f32 = jnp.float32
    src = edge_index[0].astype(jnp.int32)
    dst = edge_index[1].astype(jnp.int32)
    packed = src * (1 << SHIFT) + dst
    padrow = N + jnp.arange(EP - E, dtype=jnp.int32) % (NP - N)
    padv = padrow * (1 << SHIFT) + padrow
    edges = jnp.concatenate([packed, padv]).reshape(NW, NBLK, KB)

    x_p = jnp.zeros((NP, 128), f32).at[:N].set(x)
    bid_p = jnp.full((NP,), G, jnp.int32).at[:N].set(batch_ids.astype(jnp.int32))
    bid3 = bid_p.reshape(GRID, 1, BLK)

    b1r = b1.reshape(1, 128)
    b2r = b2.reshape(1, 128)
    b3r = b3.reshape(1, 128)
    blr = blin.reshape(1, 128)
    wo_p = jnp.zeros((128, 128), f32).at[:, :C].set(Wout)
    bo_p = jnp.zeros((1, 128), f32).at[0, :C].set(bout)

    degp = _sc_degree(edges)
    m1, dv = _t1(x_p, W1, degp[:NP], degp[NP:])
    p1_ = _sc_edge_pass(m1, edges)
    m2 = _t2(p1_[:NP], p1_[NP:], m1, dv, b1r, W2)
    p2_ = _sc_edge_pass(m2, edges)
    m3 = _t2(p2_[:NP], p2_[NP:], m2, dv, b2r, W3)
    p3_ = _sc_edge_pass(m3, edges)
    lg, pr, em = _t3(p3_[:NP], p3_[NP:], m3, dv, b3r, bid3, Wlin, blr,
                     wo_p, bo_p)
    return lg[:, :C], pr[:, :C], em
